# Initial kernel scaffold; baseline (speedup 1.0000x reference)
import jax
import jax.numpy as jnp
from jax.experimental import pallas as pl


def kernel(features, vil_input, W, b):
    raise NotImplementedError("write your pallas kernel here")


# speedup vs baseline: 12.1537x; 12.1537x over previous
import jax
import jax.numpy as jnp
from jax import lax
from jax.experimental import pallas as pl
from jax.experimental.pallas import tpu as pltpu
from jax.experimental.pallas import tpu_sc as plsc

B, C, HF, WF = 8, 128, 96, 96
HV, WV = 384, 384
K = 50
THRESH = 0.3
NEG = -1e38
NSPAN = (HV * WV) // 128

NT = 4
CPT = C // NT
NKN = 64
NCH = CPT * NKN // 128


def _detect_select_body(vil_ref, flat_ref, scr, m_scr):
    pid = pl.program_id(0)

    @pl.when(pid < B)
    def _detect():
        vil = vil_ref[0, 0]
        ninf = float('-inf')
        p = jnp.concatenate(
            [jnp.full((HV, 4), ninf), vil, jnp.full((HV, 3), ninf)],
            axis=1)
        m2 = jnp.maximum(p[:, 0:390], p[:, 1:391])
        m4 = jnp.maximum(m2[:, 0:388], m2[:, 2:390])
        mrow = jnp.maximum(m4[:, 0:WV], m4[:, 4:WV + 4])
        q = jnp.concatenate(
            [jnp.full((4, WV), ninf), mrow, jnp.full((3, WV), ninf)],
            axis=0)
        n2 = jnp.maximum(q[0:390, :], q[1:391, :])
        n4 = jnp.maximum(n2[0:388, :], n2[2:390, :])
        mfull = jnp.maximum(n4[0:HV, :], n4[4:HV + 4, :])

        peaks = jnp.logical_and(vil > THRESH, mfull == vil)
        scores = jnp.where(peaks, vil, NEG)
        scr[pl.ds(pid, 1)] = scores.reshape(1, HV, WV)
        m_scr[pl.ds(pid, 1), :] = jnp.max(
            scores.reshape(NSPAN, 128), axis=1).reshape(1, NSPAN)

    @pl.when(pid == B)
    def _select():
        m0 = m_scr[:, :]
        sidx = lax.broadcasted_iota(jnp.int32, (B, NSPAN), 1)
        lane384 = lax.broadcasted_iota(jnp.int32, (B, WV), 1)
        liota = lax.broadcasted_iota(jnp.int32, (1, 128), 1)
        big = jnp.int32(1 << 30)
        ninf = float('-inf')

        def body(k, carry):
            m, acc = carry
            gm = jnp.max(m, axis=1, keepdims=True)
            sid = jnp.min(jnp.where(m == gm, sidx, big),
                          axis=1, keepdims=True)
            y = sid // 3
            c = sid - 3 * y
            rows = jnp.concatenate(
                [scr[b, pl.ds(y[b, 0], 1), :] for b in range(B)], axis=0)
            inspan = (lane384 >= c * 128) & (lane384 < c * 128 + 128)
            xabs = jnp.min(
                jnp.where(inspan & (rows == gm), lane384, big),
                axis=1, keepdims=True)
            flat = y * WV + xabs
            rows2 = jnp.where(lane384 == xabs, ninf, rows)
            for b in range(B):
                scr[b, pl.ds(y[b, 0], 1), :] = rows2[b:b + 1, :]
            smax = jnp.max(jnp.where(inspan, rows2, ninf),
                           axis=1, keepdims=True)
            mn = jnp.where(sidx == sid, smax, m)
            accn = jnp.where(liota == k, flat, acc)
            return mn, accn

        _, acc = lax.fori_loop(
            0, K, body, (m0, jnp.zeros((B, 128), jnp.int32)))
        flat_ref[:, 0, :] = acc


def _sc_gather_body(feat_ref, flat_ref, bases_ref, out_ref,
                    pos_v, base_v, idx_v, dat_v, sem):
    cid = lax.axis_index("c")
    sid = lax.axis_index("s")
    wid = sid * 2 + cid
    b = wid // NT
    q = wid - b * NT

    pltpu.sync_copy(flat_ref.at[b, 0, pl.ds(0, NKN)], pos_v)
    d384 = jnp.full((16,), WV, jnp.int32)
    d4 = jnp.full((16,), 4, jnp.int32)
    for v in range(NKN // 16):
        f = pos_v[pl.ds(v * 16, 16)]
        yv = lax.div(f, d384)
        xv = f - yv * WV
        pos_v[pl.ds(v * 16, 16)] = (lax.div(yv, d4) * WF + lax.div(xv, d4))

    pltpu.sync_copy(bases_ref.at[wid], base_v)
    bv = base_v[...]
    for c in range(CPT):
        cof = bv + c * (HF * WF)
        for kk in range(NKN // 16):
            g0 = c * NKN + kk * 16
            idx_v[g0 // 128, pl.ds(g0 % 128, 16)] = (
                cof + pos_v[pl.ds(kk * 16, 16)])

    copies = [
        pltpu.make_async_copy(feat_ref.at[idx_v.at[j]], dat_v.at[j], sem)
        for j in range(NCH)
    ]
    for cp in copies:
        cp.start()
    for cp in copies:
        cp.wait()

    pltpu.sync_copy(dat_v, out_ref.at[b, q])


def _proj_body(g_ref, w_ref, b_ref, out_ref):
    gt = g_ref[0]
    node = lax.dot_general(gt, w_ref[:, :], (((0,), (1,)), ((), ())),
                           preferred_element_type=jnp.float32) + b_ref[:, :]
    out_ref[0] = node


def kernel(features, vil_input, W, b):
    b2 = b.reshape(1, C)

    flat3 = pl.pallas_call(
        _detect_select_body,
        grid=(B + 1,),
        in_specs=[
            pl.BlockSpec((1, 1, HV, WV),
                         lambda i: (jnp.minimum(i, B - 1), 11, 0, 0)),
        ],
        out_specs=pl.BlockSpec((B, 1, 128), lambda i: (0, 0, 0)),
        out_shape=jax.ShapeDtypeStruct((B, 1, 128), jnp.int32),
        scratch_shapes=[
            pltpu.VMEM((B, HV, WV), jnp.float32),
            pltpu.VMEM((B, NSPAN), jnp.float32),
        ],
        compiler_params=pltpu.CompilerParams(
            dimension_semantics=("arbitrary",)),
    )(vil_input)

    sc_gather = pl.kernel(
        _sc_gather_body,
        out_type=jax.ShapeDtypeStruct((B, NT, NCH, 128), jnp.float32),
        mesh=plsc.VectorSubcoreMesh(core_axis_name="c", subcore_axis_name="s"),
        scratch_types=[
            pltpu.VMEM((NKN,), jnp.int32),
            pltpu.VMEM((16,), jnp.int32),
            pltpu.VMEM((NCH, 128), jnp.int32),
            pltpu.VMEM((NCH, 128), jnp.float32),
            pltpu.SemaphoreType.DMA,
        ],
    )
    widv = jnp.arange(32, dtype=jnp.int32)
    bases = jnp.repeat(
        (((widv // NT) * C + (widv % NT) * CPT) * (HF * WF))[:, None],
        16, axis=1)
    strips = sc_gather(features.reshape(B * C * HF * WF), flat3, bases)

    nodes_pad = pl.pallas_call(
        _proj_body,
        grid=(B,),
        in_specs=[
            pl.BlockSpec((1, C, NKN), lambda i: (i, 0, 0)),
            pl.BlockSpec((C, C), lambda i: (0, 0)),
            pl.BlockSpec((1, C), lambda i: (0, 0)),
        ],
        out_specs=pl.BlockSpec((1, NKN, C), lambda i: (i, 0, 0)),
        out_shape=jax.ShapeDtypeStruct((B, NKN, C), jnp.float32),
        compiler_params=pltpu.CompilerParams(
            dimension_semantics=("arbitrary",)),
    )(strips.reshape(B, C, NKN), W, b2)
    node_features = nodes_pad[:, :K, :].reshape(B * K, C)

    flat = flat3[:, 0, :K]
    yv = flat // WV
    xv = flat % WV
    node_positions = jnp.stack([yv // 4, xv // 4], axis=-1).astype(
        jnp.float32).reshape(B * K, 2)
    batch_idx = jnp.repeat(jnp.arange(B, dtype=jnp.int32), K)
    return node_features, node_positions, batch_idx
